# Initial kernel scaffold; baseline (speedup 1.0000x reference)
import jax
import jax.numpy as jnp
from jax.experimental import pallas as pl


def kernel(z, codebook):
    raise NotImplementedError("write your pallas kernel here")


# speedup vs baseline: 2.0597x; 2.0597x over previous
import functools

import jax
import jax.numpy as jnp
from jax import lax
from jax.experimental import pallas as pl
from jax.experimental.pallas import tpu as pltpu
from jax.experimental.pallas import tpu_sc as plsc

CB = 8192
D = 256
TM = 256
NC = 2
NS = 16
NW = NC * NS
COMMIT = 0.25


def _dist_body(zsq_ref, csq_ref, flat_ref, cb_ref, dist_ref, idx_ref, minv_ref):
    flat = flat_ref[...]
    cb = cb_ref[...]
    dot = lax.dot_general(flat, cb, (((1,), (1,)), ((), ())),
                          preferred_element_type=jnp.float32)
    d = (zsq_ref[...] - 2.0 * dot) + csq_ref[...]
    dist_ref[...] = d
    minv = jnp.min(d, axis=1, keepdims=True)
    cols = lax.broadcasted_iota(jnp.int32, (TM, CB), 1)
    idx = jnp.min(jnp.where(d == minv, cols, CB), axis=1)
    idx_ref[...] = idx.astype(jnp.int32)
    minv_ref[...] = minv[:, 0]


def _distances_pallas(zsq, csq, flat, codebook):
    return pl.pallas_call(
        _dist_body,
        grid=(CB // TM,),
        in_specs=[
            pl.BlockSpec((TM, 1), lambda m: (m, 0)),
            pl.BlockSpec((1, CB), lambda m: (0, 0)),
            pl.BlockSpec((TM, D), lambda m: (m, 0)),
            pl.BlockSpec((CB, D), lambda m: (0, 0)),
        ],
        out_specs=[
            pl.BlockSpec((TM, CB), lambda m: (m, 0)),
            pl.BlockSpec((TM,), lambda m: (m,)),
            pl.BlockSpec((TM,), lambda m: (m,)),
        ],
        out_shape=[
            jax.ShapeDtypeStruct((CB, CB), jnp.float32),
            jax.ShapeDtypeStruct((CB,), jnp.int32),
            jax.ShapeDtypeStruct((CB,), jnp.float32),
        ],
        compiler_params=pltpu.CompilerParams(
            vmem_limit_bytes=100 * 1024 * 1024,
        ),
    )(zsq, csq, flat, codebook)


BPW = CB // NW


def _sc_body(cb_hbm, idx_hbm, q_hbm, counts_hbm, idx_v, rows_v, counts_v, sem):
    wid = lax.axis_index("s") * NC + lax.axis_index("c")
    base = wid * BPW
    pltpu.sync_copy(idx_hbm.at[pl.ds(base, BPW)], idx_v)
    pltpu.async_copy(cb_hbm.at[idx_v], rows_v, sem).wait()
    pltpu.sync_copy(rows_v, q_hbm.at[pl.ds(base, BPW)])

    def _zero(i, carry):
        counts_v[pl.ds(i * 16, 16)] = jnp.zeros((16,), jnp.float32)
        return carry
    lax.fori_loop(0, CB // 16, _zero, 0)

    ones = jnp.ones((16,), jnp.float32)

    def _hist(i, carry):
        iv = idx_v[pl.ds(i * 16, 16)]
        plsc.addupdate_scatter(counts_v, [iv], ones)
        return carry
    lax.fori_loop(0, BPW // 16, _hist, 0)
    pltpu.sync_copy(counts_v, counts_hbm.at[wid])


@functools.cache
def _sc_gather_hist():
    return pl.kernel(
        _sc_body,
        out_type=[
            jax.ShapeDtypeStruct((CB, D), jnp.float32),
            jax.ShapeDtypeStruct((NW, CB), jnp.float32),
        ],
        mesh=plsc.VectorSubcoreMesh(core_axis_name="c", subcore_axis_name="s"),
        compiler_params=pltpu.CompilerParams(needs_layout_passes=False),
        scratch_types=[
            pltpu.VMEM((BPW,), jnp.int32),
            pltpu.VMEM((BPW, D), jnp.float32),
            pltpu.VMEM((CB,), jnp.float32),
            pltpu.SemaphoreType.DMA,
        ],
    )


def kernel(z, codebook):
    B, N, _ = z.shape
    flat = z.reshape(-1, D)
    zsq = jnp.sum(flat ** 2, axis=1, keepdims=True)
    csq = jnp.sum(codebook ** 2, axis=1)[None, :]

    distances, indices, minvals = _distances_pallas(zsq, csq, flat, codebook)
    quantized, partial_counts = _sc_gather_hist()(codebook, indices)

    z_q = z + lax.stop_gradient(quantized.reshape(z.shape) - z)
    codebook_loss = jnp.sum(minvals) / (CB * D)
    commit_loss = COMMIT * codebook_loss
    counts = jnp.sum(partial_counts, axis=0)
    avg_probs = counts / CB
    perplexity = jnp.exp(-jnp.sum(avg_probs * jnp.log(avg_probs + 1e-10)))
    return (z_q,
            indices.reshape(B, N),
            commit_loss,
            codebook_loss,
            perplexity,
            distances.reshape(B, N, CB))


# speedup vs baseline: 2.0749x; 1.0074x over previous
import functools

import jax
import jax.numpy as jnp
from jax import lax
from jax.experimental import pallas as pl
from jax.experimental.pallas import tpu as pltpu
from jax.experimental.pallas import tpu_sc as plsc

CB = 8192
D = 256
TM = 256
FM = 1024
NC = 2
NS = 16
NW = NC * NS
BPW = CB // NW
COMMIT = 0.25


def _dist_body(zsq_ref, csq_ref, flat_ref, cb_ref, dist_ref, idx_ref):
    flat = flat_ref[...]
    cb = cb_ref[...]
    dot = lax.dot_general(flat, cb, (((1,), (1,)), ((), ())),
                          preferred_element_type=jnp.float32)
    d = (zsq_ref[...] - 2.0 * dot) + csq_ref[...]
    dist_ref[...] = d
    idx_ref[...] = jnp.argmin(d, axis=1).astype(jnp.int32)


def _distances_pallas(zsq, csq, flat, codebook):
    return pl.pallas_call(
        _dist_body,
        grid=(CB // TM,),
        in_specs=[
            pl.BlockSpec((TM, 1), lambda m: (m, 0)),
            pl.BlockSpec((1, CB), lambda m: (0, 0)),
            pl.BlockSpec((TM, D), lambda m: (m, 0)),
            pl.BlockSpec((CB, D), lambda m: (0, 0)),
        ],
        out_specs=[
            pl.BlockSpec((TM, CB), lambda m: (m, 0)),
            pl.BlockSpec((TM,), lambda m: (m,)),
        ],
        out_shape=[
            jax.ShapeDtypeStruct((CB, CB), jnp.float32),
            jax.ShapeDtypeStruct((CB,), jnp.int32),
        ],
        compiler_params=pltpu.CompilerParams(
            vmem_limit_bytes=100 * 1024 * 1024,
        ),
    )(zsq, csq, flat, codebook)


def _sc_body(cb_hbm, idx_hbm, q_hbm, counts_hbm, idx_v, rows_v, counts_v, sem):
    wid = lax.axis_index("s") * NC + lax.axis_index("c")
    base = wid * BPW
    pltpu.sync_copy(idx_hbm.at[pl.ds(base, BPW)], idx_v)
    pltpu.async_copy(cb_hbm.at[idx_v], rows_v, sem).wait()
    pltpu.sync_copy(rows_v, q_hbm.at[pl.ds(base, BPW)])

    def _zero(i, carry):
        counts_v[pl.ds(i * 16, 16)] = jnp.zeros((16,), jnp.float32)
        return carry
    lax.fori_loop(0, CB // 16, _zero, 0)

    ones = jnp.ones((16,), jnp.float32)

    def _hist(i, carry):
        iv = idx_v[pl.ds(i * 16, 16)]
        plsc.addupdate_scatter(counts_v, [iv], ones)
        return carry
    lax.fori_loop(0, BPW // 16, _hist, 0)
    pltpu.sync_copy(counts_v, counts_hbm.at[wid])


@functools.cache
def _sc_gather_hist():
    return pl.kernel(
        _sc_body,
        out_type=[
            jax.ShapeDtypeStruct((CB, D), jnp.float32),
            jax.ShapeDtypeStruct((NW, CB), jnp.float32),
        ],
        mesh=plsc.VectorSubcoreMesh(core_axis_name="c", subcore_axis_name="s"),
        compiler_params=pltpu.CompilerParams(needs_layout_passes=False),
        scratch_types=[
            pltpu.VMEM((BPW,), jnp.int32),
            pltpu.VMEM((BPW, D), jnp.float32),
            pltpu.VMEM((CB,), jnp.float32),
            pltpu.SemaphoreType.DMA,
        ],
    )


def _finish_body(z_ref, q_ref, zq_ref, rowsq_ref):
    zv = z_ref[...]
    qv = q_ref[...]
    diff = qv - zv
    zq_ref[...] = zv + diff
    rowsq_ref[...] = jnp.sum(diff * diff, axis=1)


def _finish_pallas(flat, quantized):
    return pl.pallas_call(
        _finish_body,
        grid=(CB // FM,),
        in_specs=[
            pl.BlockSpec((FM, D), lambda m: (m, 0)),
            pl.BlockSpec((FM, D), lambda m: (m, 0)),
        ],
        out_specs=[
            pl.BlockSpec((FM, D), lambda m: (m, 0)),
            pl.BlockSpec((FM,), lambda m: (m,)),
        ],
        out_shape=[
            jax.ShapeDtypeStruct((CB, D), jnp.float32),
            jax.ShapeDtypeStruct((CB,), jnp.float32),
        ],
    )(flat, quantized)


def kernel(z, codebook):
    B, N, _ = z.shape
    flat = z.reshape(-1, D)
    zsq = jnp.sum(flat ** 2, axis=1, keepdims=True)
    csq = jnp.sum(codebook ** 2, axis=1)[None, :]

    distances, indices = _distances_pallas(zsq, csq, flat, codebook)
    quantized, partial_counts = _sc_gather_hist()(codebook, indices)
    zq_flat, rowsq = _finish_pallas(flat, quantized)

    z_q = zq_flat.reshape(z.shape)
    codebook_loss = jnp.sum(rowsq) / (CB * D)
    commit_loss = COMMIT * codebook_loss
    counts = jnp.sum(partial_counts, axis=0)
    avg_probs = counts / CB
    perplexity = jnp.exp(-jnp.sum(avg_probs * jnp.log(avg_probs + 1e-10)))
    return (z_q,
            indices.reshape(B, N),
            commit_loss,
            codebook_loss,
            perplexity,
            distances.reshape(B, N, CB))
